# baseline (speedup 1.0000x reference)
import functools

import jax
import jax.numpy as jnp
from jax import lax
from jax.experimental import pallas as pl
from jax.experimental.pallas import tpu as pltpu
from jax.experimental.pallas import tpu_sc as plsc

NCLS = 21
B, C, H, W = 4, 96, 384, 384
HW = H * W
NSAMP = int(HW * 0.8)
NH = 343
NKEEP = NH * NH
NTILES = 16
CH = 7424
NP = NTILES * CH


def _al(x):
    return pl.multiple_of(x, 128)


def _gather_body(f_hbm, t_hbm, idx_hbm, out_f, out_t,
                 idx_v, vals_v, tvals_v, row_sh, trow_sh):
    cid = lax.axis_index("c")
    sid = lax.axis_index("s")

    for j in range(2):
        b = cid * 2 + j
        pltpu.sync_copy(idx_hbm.at[pl.ds(_al(b * NP + sid * CH), CH)], idx_v)

        @pl.when(sid == 0)
        def _():
            pltpu.sync_copy(t_hbm.at[pl.ds(_al(b * HW), HW)], trow_sh)
        plsc.subcore_barrier()
        pltpu.sync_copy(trow_sh.at[idx_v], tvals_v)
        pltpu.sync_copy(tvals_v, out_t.at[pl.ds(_al(b * NP + sid * CH), CH)])

        def chan(c, carry):
            @pl.when(sid == 0)
            def _():
                pltpu.sync_copy(f_hbm.at[pl.ds(_al((b * C + c) * HW), HW)],
                                row_sh)
            plsc.subcore_barrier()
            pltpu.sync_copy(row_sh.at[idx_v], vals_v)
            pltpu.sync_copy(
                vals_v,
                out_f.at[pl.ds(_al((b * C + c) * NP + sid * CH), CH)])
            plsc.subcore_barrier()
            return carry

        lax.fori_loop(0, C, chan, 0)


def _sample_gather(f_flat1, t_flat1, idx1):
    mesh = plsc.VectorSubcoreMesh(core_axis_name="c", subcore_axis_name="s")
    run = pl.kernel(
        _gather_body,
        mesh=mesh,
        out_type=(
            jax.ShapeDtypeStruct((B * C * NP,), jnp.float32),
            jax.ShapeDtypeStruct((B * NP,), jnp.int32),
        ),
        scratch_types=[
            pltpu.VMEM((CH,), jnp.int32),
            pltpu.VMEM((CH,), jnp.float32),
            pltpu.VMEM((CH,), jnp.int32),
            pltpu.VMEM_SHARED((HW,), jnp.float32),
            pltpu.VMEM_SHARED((HW,), jnp.int32),
        ],
    )
    return run(f_flat1, t_flat1, idx1)


def kernel(features, targets, sampling_weights):
    total = float(targets.size)
    flat_t = targets.reshape(-1)
    counts = jnp.bincount(flat_t, length=NCLS)
    num_classes = jnp.sum(counts > 0).astype(jnp.float32)
    counts_f = jnp.maximum(counts.astype(jnp.float32), 1.0)
    class_w = jnp.where(counts > 0, total / (num_classes * counts_f), 1.0)
    probs = class_w[targets].astype(jnp.float32)
    probs = probs / probs.sum()
    probs = probs * sampling_weights[0]
    p_flat = probs.reshape(B, HW)

    u = jax.random.uniform(jax.random.key(42), (B, HW), minval=1e-20, maxval=1.0)
    g = -jnp.log(-jnp.log(u))
    keys_g = jnp.log(jnp.maximum(p_flat, 1e-30)) + g

    _, idx = lax.top_k(keys_g, NSAMP)
    idx_in = jnp.pad(idx, ((0, 0), (0, NP - NSAMP))).reshape(-1)

    f_flat1 = features.reshape(-1)
    t_flat1 = targets.reshape(-1)
    out_f, out_t = _sample_gather(f_flat1, t_flat1, idx_in)
    sf = out_f.reshape(B, C, NP)[:, :, :NKEEP].reshape(B, C, NH, NH)
    st = out_t.reshape(B, NP)[:, :NKEEP].reshape(B, NH, NH)
    return sf, st


# speedup vs baseline: 1.1184x; 1.1184x over previous
import functools

import jax
import jax.numpy as jnp
from jax import lax
from jax.experimental import pallas as pl
from jax.experimental.pallas import tpu as pltpu
from jax.experimental.pallas import tpu_sc as plsc

NCLS = 21
B, C, H, W = 4, 96, 384, 384
HW = H * W
NSAMP = int(HW * 0.8)
NH = 343
NKEEP = NH * NH
NTILES = 16
CH = 7424
NP = NTILES * CH

CHK = HW // NTILES
LB = CHK // 16
RADIX = 32
NPASS = 7


def _al(x):
    return pl.multiple_of(x, 128)


def _sort_body(u_hbm, out_hbm,
               uv, iv, stage_u, stage_i, stage_d, hist_v, ctr_v, hist_all,
               totals_s, dbase_s, uA, uB, iA, iB, hist_sh):
    cid = lax.axis_index("c")
    sid = lax.axis_index("s")
    ii = lax.iota(jnp.int32, 16)
    ones = jnp.ones((16,), jnp.int32)

    def row_body(r, carry):
        b = cid * 2 + r
        for p in range(NPASS):
            shift = 5 * p
            src_u, src_i = (uB, iB) if (p % 2 == 0) else (uA, iA)
            dst_u, dst_i = (uA, iA) if (p % 2 == 0) else (uB, iB)
            if p == 0:
                pltpu.sync_copy(
                    u_hbm.at[pl.ds(_al(b * HW + sid * CHK), CHK)], uv)
            else:
                pltpu.sync_copy(src_u.at[pl.ds(sid * CHK, CHK)], uv)
                pltpu.sync_copy(src_i.at[pl.ds(sid * CHK, CHK)], iv)

            def zstep(k, c):
                hist_v[pl.ds(k * 16, 16)] = jnp.zeros((16,), jnp.int32)
                return c
            lax.fori_loop(0, RADIX, zstep, 0)

            def hstep(j, c):
                pos = ii * LB + j
                u = plsc.load_gather(uv, [pos])
                d = lax.shift_right_logical(u, shift) & (RADIX - 1)
                plsc.addupdate_scatter(hist_v, [d * 16 + ii], ones)
                return c
            lax.fori_loop(0, LB, hstep, 0)
            pltpu.sync_copy(hist_v, hist_sh.at[pl.ds(sid * 512, 512)])
            plsc.subcore_barrier()

            pltpu.sync_copy(hist_sh, hist_all)

            def scan_digit(d, c0):
                def scan_t(t, c):
                    v = hist_all[pl.ds(t * 512 + d * 16, 16)]
                    cs = plsc.cumsum(v)

                    @pl.when(t == sid)
                    def _():
                        ctr_v[pl.ds(d * 16, 16)] = cs - v + c
                    return c + jnp.sum(v)
                totals_s[d] = lax.fori_loop(0, NTILES, scan_t, 0)
                return c0
            lax.fori_loop(0, RADIX, scan_digit, 0)

            def dbase(d, c):
                dbase_s[d] = c
                return c + totals_s[d]
            lax.fori_loop(0, RADIX, dbase, 0)

            def addb(d, c):
                ctr_v[pl.ds(d * 16, 16)] = (
                    ctr_v[pl.ds(d * 16, 16)] + dbase_s[d])
                return c
            lax.fori_loop(0, RADIX, addb, 0)

            def pstep(j, c):
                pos = ii * LB + j
                u = plsc.load_gather(uv, [pos])
                if p == 0:
                    iv_val = pos + sid * CHK
                else:
                    iv_val = plsc.load_gather(iv, [pos])
                d = lax.shift_right_logical(u, shift) & (RADIX - 1)
                cp = d * 16 + ii
                old = plsc.load_gather(ctr_v, [cp])
                plsc.store_scatter(ctr_v, [cp], old + ones)
                stage_d[pl.ds(j * 16, 16)] = old
                if p < NPASS - 1:
                    stage_u[pl.ds(j * 16, 16)] = u
                stage_i[pl.ds(j * 16, 16)] = iv_val
                return c
            lax.fori_loop(0, LB, pstep, 0)
            if p < NPASS - 1:
                pltpu.sync_copy(stage_u, dst_u.at[stage_d])
            pltpu.sync_copy(stage_i, dst_i.at[stage_d])
            plsc.subcore_barrier()

        pltpu.sync_copy(iA.at[pl.ds(sid * CHK, CHK)], iv)
        pltpu.sync_copy(iv, out_hbm.at[pl.ds(_al(b * HW + sid * CHK), CHK)])
        plsc.subcore_barrier()
        return carry

    lax.fori_loop(0, 2, row_body, 0)


def _radix_sort(u_flat):
    mesh = plsc.VectorSubcoreMesh(core_axis_name="c", subcore_axis_name="s")
    run = pl.kernel(
        _sort_body,
        mesh=mesh,
        compiler_params=pltpu.CompilerParams(needs_layout_passes=False),
        out_type=jax.ShapeDtypeStruct((B * HW,), jnp.int32),
        scratch_types=[
            pltpu.VMEM((CHK,), jnp.int32),
            pltpu.VMEM((CHK,), jnp.int32),
            pltpu.VMEM((CHK,), jnp.int32),
            pltpu.VMEM((CHK,), jnp.int32),
            pltpu.VMEM((CHK,), jnp.int32),
            pltpu.VMEM((RADIX * 16,), jnp.int32),
            pltpu.VMEM((RADIX * 16,), jnp.int32),
            pltpu.VMEM((RADIX * 16 * NTILES,), jnp.int32),
            pltpu.SMEM((RADIX,), jnp.int32),
            pltpu.SMEM((RADIX,), jnp.int32),
            pltpu.VMEM_SHARED((HW,), jnp.int32),
            pltpu.VMEM_SHARED((HW,), jnp.int32),
            pltpu.VMEM_SHARED((HW,), jnp.int32),
            pltpu.VMEM_SHARED((HW,), jnp.int32),
            pltpu.VMEM_SHARED((RADIX * 16 * NTILES,), jnp.int32),
        ],
    )
    return run(u_flat)


def _gather_body(f_hbm, t_hbm, idx_hbm, out_f, out_t,
                 idx_v, vals_v, tvals_v, row_sh, trow_sh):
    cid = lax.axis_index("c")
    sid = lax.axis_index("s")

    for j in range(2):
        b = cid * 2 + j
        pltpu.sync_copy(idx_hbm.at[pl.ds(_al(b * NP + sid * CH), CH)], idx_v)

        @pl.when(sid == 0)
        def _():
            pltpu.sync_copy(t_hbm.at[pl.ds(_al(b * HW), HW)], trow_sh)
        plsc.subcore_barrier()
        pltpu.sync_copy(trow_sh.at[idx_v], tvals_v)
        pltpu.sync_copy(tvals_v, out_t.at[pl.ds(_al(b * NP + sid * CH), CH)])

        def chan(c, carry):
            @pl.when(sid == 0)
            def _():
                pltpu.sync_copy(f_hbm.at[pl.ds(_al((b * C + c) * HW), HW)],
                                row_sh)
            plsc.subcore_barrier()
            pltpu.sync_copy(row_sh.at[idx_v], vals_v)
            pltpu.sync_copy(
                vals_v,
                out_f.at[pl.ds(_al((b * C + c) * NP + sid * CH), CH)])
            plsc.subcore_barrier()
            return carry

        lax.fori_loop(0, C, chan, 0)


def _sample_gather(f_flat1, t_flat1, idx1):
    mesh = plsc.VectorSubcoreMesh(core_axis_name="c", subcore_axis_name="s")
    run = pl.kernel(
        _gather_body,
        mesh=mesh,
        out_type=(
            jax.ShapeDtypeStruct((B * C * NP,), jnp.float32),
            jax.ShapeDtypeStruct((B * NP,), jnp.int32),
        ),
        scratch_types=[
            pltpu.VMEM((CH,), jnp.int32),
            pltpu.VMEM((CH,), jnp.float32),
            pltpu.VMEM((CH,), jnp.int32),
            pltpu.VMEM_SHARED((HW,), jnp.float32),
            pltpu.VMEM_SHARED((HW,), jnp.int32),
        ],
    )
    return run(f_flat1, t_flat1, idx1)


def kernel(features, targets, sampling_weights):
    total = float(targets.size)
    flat_t = targets.reshape(-1)
    counts = jnp.bincount(flat_t, length=NCLS)
    num_classes = jnp.sum(counts > 0).astype(jnp.float32)
    counts_f = jnp.maximum(counts.astype(jnp.float32), 1.0)
    class_w = jnp.where(counts > 0, total / (num_classes * counts_f), 1.0)
    probs = class_w[targets].astype(jnp.float32)
    probs = probs / probs.sum()
    probs = probs * sampling_weights[0]
    p_flat = probs.reshape(B, HW)

    u = jax.random.uniform(jax.random.key(42), (B, HW), minval=1e-20, maxval=1.0)
    g = -jnp.log(-jnp.log(u))
    keys_g = jnp.log(jnp.maximum(p_flat, 1e-30)) + g

    kb = lax.bitcast_convert_type(keys_g, jnp.int32)
    u_desc = jnp.where(kb >= 0, ~(kb ^ jnp.int32(-2**31)), kb)
    sorted_idx = _radix_sort(u_desc.reshape(-1))
    idx_in = sorted_idx.reshape(B, HW)[:, :NP].reshape(-1)

    f_flat1 = features.reshape(-1)
    t_flat1 = targets.reshape(-1)
    out_f, out_t = _sample_gather(f_flat1, t_flat1, idx_in)
    sf = out_f.reshape(B, C, NP)[:, :, :NKEEP].reshape(B, C, NH, NH)
    st = out_t.reshape(B, NP)[:, :NKEEP].reshape(B, NH, NH)
    return sf, st


# speedup vs baseline: 1.5060x; 1.3465x over previous
import functools

import jax
import jax.numpy as jnp
from jax import lax
from jax.experimental import pallas as pl
from jax.experimental.pallas import tpu as pltpu
from jax.experimental.pallas import tpu_sc as plsc

NCLS = 21
B, C, H, W = 4, 96, 384, 384
HW = H * W
NSAMP = int(HW * 0.8)
NH = 343
NKEEP = NH * NH
NTILES = 16
CH = 7424
NP = NTILES * CH

CHK = HW // NTILES
LB = CHK // 16
RADIX = 32
NPASS = 7


def _al(x):
    return pl.multiple_of(x, 128)


def _sort_body(u_hbm, out_hbm,
               uv, iv, stage_u, stage_i, stage_d, hist_v, ctr_v, hist_all,
               totals_s, dbase_s, uA, uB, iA, iB, hist_sh):
    cid = lax.axis_index("c")
    sid = lax.axis_index("s")
    ii = lax.iota(jnp.int32, 16)
    ones = jnp.ones((16,), jnp.int32)

    def row_body(r, carry):
        b = cid * 2 + r
        for p in range(NPASS):
            shift = 5 * p
            src_u, src_i = (uB, iB) if (p % 2 == 0) else (uA, iA)
            dst_u, dst_i = (uA, iA) if (p % 2 == 0) else (uB, iB)
            if p == 0:
                pltpu.sync_copy(
                    u_hbm.at[pl.ds(_al(b * HW + sid * CHK), CHK)], uv)
            else:
                pltpu.sync_copy(src_u.at[pl.ds(sid * CHK, CHK)], uv)
                pltpu.sync_copy(src_i.at[pl.ds(sid * CHK, CHK)], iv)

            def zstep(k, c):
                hist_v[pl.ds(k * 16, 16)] = jnp.zeros((16,), jnp.int32)
                return c
            lax.fori_loop(0, RADIX, zstep, 0)

            def hstep(j, c):
                pos = ii * LB + j
                u = plsc.load_gather(uv, [pos])
                d = lax.shift_right_logical(u, shift) & (RADIX - 1)
                plsc.addupdate_scatter(hist_v, [d * 16 + ii], ones)
                return c
            lax.fori_loop(0, LB, hstep, 0)
            pltpu.sync_copy(hist_v, hist_sh.at[pl.ds(sid * 512, 512)])
            plsc.subcore_barrier()

            pltpu.sync_copy(hist_sh, hist_all)

            def scan_digit(d, c0):
                def scan_t(t, c):
                    v = hist_all[pl.ds(t * 512 + d * 16, 16)]
                    cs = plsc.cumsum(v)

                    @pl.when(t == sid)
                    def _():
                        ctr_v[pl.ds(d * 16, 16)] = cs - v + c
                    return c + jnp.sum(v)
                totals_s[d] = lax.fori_loop(0, NTILES, scan_t, 0)
                return c0
            lax.fori_loop(0, RADIX, scan_digit, 0)

            def dbase(d, c):
                dbase_s[d] = c
                return c + totals_s[d]
            lax.fori_loop(0, RADIX, dbase, 0)

            def addb(d, c):
                ctr_v[pl.ds(d * 16, 16)] = (
                    ctr_v[pl.ds(d * 16, 16)] + dbase_s[d])
                return c
            lax.fori_loop(0, RADIX, addb, 0)

            def pstep(j, c):
                pos = ii * LB + j
                u = plsc.load_gather(uv, [pos])
                if p == 0:
                    iv_val = pos + sid * CHK
                else:
                    iv_val = plsc.load_gather(iv, [pos])
                d = lax.shift_right_logical(u, shift) & (RADIX - 1)
                cp = d * 16 + ii
                old = plsc.load_gather(ctr_v, [cp])
                plsc.store_scatter(ctr_v, [cp], old + ones)
                stage_d[pl.ds(j * 16, 16)] = old
                if p < NPASS - 1:
                    stage_u[pl.ds(j * 16, 16)] = u
                stage_i[pl.ds(j * 16, 16)] = iv_val
                return c
            lax.fori_loop(0, LB, pstep, 0)
            if p < NPASS - 1:
                pltpu.sync_copy(stage_u, dst_u.at[stage_d])
            pltpu.sync_copy(stage_i, dst_i.at[stage_d])
            plsc.subcore_barrier()

        pltpu.sync_copy(iA.at[pl.ds(sid * CHK, CHK)], iv)
        pltpu.sync_copy(iv, out_hbm.at[pl.ds(_al(b * HW + sid * CHK), CHK)])
        plsc.subcore_barrier()
        return carry

    lax.fori_loop(0, 2, row_body, 0)


def _radix_sort(u_flat):
    mesh = plsc.VectorSubcoreMesh(core_axis_name="c", subcore_axis_name="s")
    run = pl.kernel(
        _sort_body,
        mesh=mesh,
        compiler_params=pltpu.CompilerParams(needs_layout_passes=False),
        out_type=jax.ShapeDtypeStruct((B * HW,), jnp.int32),
        scratch_types=[
            pltpu.VMEM((CHK,), jnp.int32),
            pltpu.VMEM((CHK,), jnp.int32),
            pltpu.VMEM((CHK,), jnp.int32),
            pltpu.VMEM((CHK,), jnp.int32),
            pltpu.VMEM((CHK,), jnp.int32),
            pltpu.VMEM((RADIX * 16,), jnp.int32),
            pltpu.VMEM((RADIX * 16,), jnp.int32),
            pltpu.VMEM((RADIX * 16 * NTILES,), jnp.int32),
            pltpu.SMEM((RADIX,), jnp.int32),
            pltpu.SMEM((RADIX,), jnp.int32),
            pltpu.VMEM_SHARED((HW,), jnp.int32),
            pltpu.VMEM_SHARED((HW,), jnp.int32),
            pltpu.VMEM_SHARED((HW,), jnp.int32),
            pltpu.VMEM_SHARED((HW,), jnp.int32),
            pltpu.VMEM_SHARED((RADIX * 16 * NTILES,), jnp.int32),
        ],
    )
    return run(u_flat)


def _gather_body(f_hbm, t_hbm, idx_hbm, out_f, out_t,
                 idx_v, vals_v, tvals_v, row_sh, trow_sh):
    cid = lax.axis_index("c")
    sid = lax.axis_index("s")

    for j in range(2):
        b = cid * 2 + j
        pltpu.sync_copy(idx_hbm.at[pl.ds(_al(b * NP + sid * CH), CH)], idx_v)

        @pl.when(sid == 0)
        def _():
            pltpu.sync_copy(t_hbm.at[pl.ds(_al(b * HW), HW)], trow_sh)
        plsc.subcore_barrier()
        pltpu.sync_copy(trow_sh.at[idx_v], tvals_v)
        pltpu.sync_copy(tvals_v, out_t.at[pl.ds(_al(b * NP + sid * CH), CH)])

        def chan(c, carry):
            @pl.when(sid == 0)
            def _():
                pltpu.sync_copy(f_hbm.at[pl.ds(_al((b * C + c) * HW), HW)],
                                row_sh)
            plsc.subcore_barrier()
            pltpu.sync_copy(row_sh.at[idx_v], vals_v)
            pltpu.sync_copy(
                vals_v,
                out_f.at[pl.ds(_al((b * C + c) * NP + sid * CH), CH)])
            plsc.subcore_barrier()
            return carry

        lax.fori_loop(0, C, chan, 0)


def _sample_gather(f_flat1, t_flat1, idx1):
    mesh = plsc.VectorSubcoreMesh(core_axis_name="c", subcore_axis_name="s")
    run = pl.kernel(
        _gather_body,
        mesh=mesh,
        out_type=(
            jax.ShapeDtypeStruct((B * C * NP,), jnp.float32),
            jax.ShapeDtypeStruct((B * NP,), jnp.int32),
        ),
        scratch_types=[
            pltpu.VMEM((CH,), jnp.int32),
            pltpu.VMEM((CH,), jnp.float32),
            pltpu.VMEM((CH,), jnp.int32),
            pltpu.VMEM_SHARED((HW,), jnp.float32),
            pltpu.VMEM_SHARED((HW,), jnp.int32),
        ],
    )
    return run(f_flat1, t_flat1, idx1)


from jax.experimental.compute_on import compute_on


@compute_on("tpu_sparsecore")
@jax.jit
def _sc_probs(cw, t):
    return cw[t].astype(jnp.float32)


def kernel(features, targets, sampling_weights):
    total = float(targets.size)
    flat_t = targets.reshape(-1)
    counts = jnp.bincount(flat_t, length=NCLS)
    num_classes = jnp.sum(counts > 0).astype(jnp.float32)
    counts_f = jnp.maximum(counts.astype(jnp.float32), 1.0)
    class_w = jnp.where(counts > 0, total / (num_classes * counts_f), 1.0)
    probs = _sc_probs(class_w, targets)
    probs = probs / probs.sum()
    probs = probs * sampling_weights[0]
    p_flat = probs.reshape(B, HW)

    u = jax.random.uniform(jax.random.key(42), (B, HW), minval=1e-20, maxval=1.0)
    g = -jnp.log(-jnp.log(u))
    keys_g = jnp.log(jnp.maximum(p_flat, 1e-30)) + g

    kb = lax.bitcast_convert_type(keys_g, jnp.int32)
    u_desc = jnp.where(kb >= 0, ~(kb ^ jnp.int32(-2**31)), kb)
    sorted_idx = _radix_sort(u_desc.reshape(-1))
    idx_in = sorted_idx.reshape(B, HW)[:, :NP].reshape(-1)

    f_flat1 = features.reshape(-1)
    t_flat1 = targets.reshape(-1)
    out_f, out_t = _sample_gather(f_flat1, t_flat1, idx_in)
    sf = out_f.reshape(B, C, NP)[:, :, :NKEEP].reshape(B, C, NH, NH)
    st = out_t.reshape(B, NP)[:, :NKEEP].reshape(B, NH, NH)
    return sf, st


# speedup vs baseline: 2.9151x; 1.9357x over previous
import functools

import jax
import jax.numpy as jnp
from jax import lax
from jax.experimental import pallas as pl
from jax.experimental.pallas import tpu as pltpu
from jax.experimental.pallas import tpu_sc as plsc

NCLS = 21
B, C, H, W = 4, 96, 384, 384
HW = H * W
NSAMP = int(HW * 0.8)
NH = 343
NKEEP = NH * NH
NTILES = 16
CH = 7424
NP = NTILES * CH

CHK = HW // NTILES
LB = CHK // 16
RADIX = 32
NPASS = 7


def _al(x):
    return pl.multiple_of(x, 128)


def _sort_body(u_hbm, out_hbm,
               uv, iv, stage_u, stage_i, stage_d, hist_v, ctr_v, hist_all,
               totals_s, dbase_s, uA, uB, iA, iB, hist_sh):
    cid = lax.axis_index("c")
    sid = lax.axis_index("s")
    ii = lax.iota(jnp.int32, 16)
    ones = jnp.ones((16,), jnp.int32)

    def row_body(r, carry):
        b = cid * 2 + r
        for p in range(NPASS):
            shift = 5 * p
            src_u, src_i = (uB, iB) if (p % 2 == 0) else (uA, iA)
            dst_u, dst_i = (uA, iA) if (p % 2 == 0) else (uB, iB)
            if p == 0:
                pltpu.sync_copy(
                    u_hbm.at[pl.ds(_al(b * HW + sid * CHK), CHK)], uv)
            else:
                pltpu.sync_copy(src_u.at[pl.ds(sid * CHK, CHK)], uv)
                pltpu.sync_copy(src_i.at[pl.ds(sid * CHK, CHK)], iv)

            def zstep(k, c):
                hist_v[pl.ds(k * 16, 16)] = jnp.zeros((16,), jnp.int32)
                return c
            lax.fori_loop(0, RADIX, zstep, 0)

            def hstep(j, c):
                pos = ii * LB + j
                u = plsc.load_gather(uv, [pos])
                d = lax.shift_right_logical(u, shift) & (RADIX - 1)
                plsc.addupdate_scatter(hist_v, [d * 16 + ii], ones)
                return c
            lax.fori_loop(0, LB, hstep, 0)
            pltpu.sync_copy(hist_v, hist_sh.at[pl.ds(sid * 512, 512)])
            plsc.subcore_barrier()

            pltpu.sync_copy(hist_sh, hist_all)

            def scan_digit(d, c0):
                def scan_t(t, c):
                    v = hist_all[pl.ds(t * 512 + d * 16, 16)]
                    cs = plsc.cumsum(v)

                    @pl.when(t == sid)
                    def _():
                        ctr_v[pl.ds(d * 16, 16)] = cs - v + c
                    return c + jnp.sum(v)
                totals_s[d] = lax.fori_loop(0, NTILES, scan_t, 0)
                return c0
            lax.fori_loop(0, RADIX, scan_digit, 0)

            def dbase(d, c):
                dbase_s[d] = c
                return c + totals_s[d]
            lax.fori_loop(0, RADIX, dbase, 0)

            def addb(d, c):
                ctr_v[pl.ds(d * 16, 16)] = (
                    ctr_v[pl.ds(d * 16, 16)] + dbase_s[d])
                return c
            lax.fori_loop(0, RADIX, addb, 0)

            def pstep(j, c):
                pos = ii * LB + j
                u = plsc.load_gather(uv, [pos])
                if p == 0:
                    iv_val = pos + sid * CHK
                else:
                    iv_val = plsc.load_gather(iv, [pos])
                d = lax.shift_right_logical(u, shift) & (RADIX - 1)
                cp = d * 16 + ii
                old = plsc.load_gather(ctr_v, [cp])
                plsc.store_scatter(ctr_v, [cp], old + ones)
                stage_d[pl.ds(j * 16, 16)] = old
                if p < NPASS - 1:
                    stage_u[pl.ds(j * 16, 16)] = u
                stage_i[pl.ds(j * 16, 16)] = iv_val
                return c
            lax.fori_loop(0, LB, pstep, 0)
            if p < NPASS - 1:
                pltpu.sync_copy(stage_u, dst_u.at[stage_d])
            pltpu.sync_copy(stage_i, dst_i.at[stage_d])
            plsc.subcore_barrier()

        pltpu.sync_copy(iA.at[pl.ds(sid * CHK, CHK)], iv)
        pltpu.sync_copy(iv, out_hbm.at[pl.ds(_al(b * HW + sid * CHK), CHK)])
        plsc.subcore_barrier()
        return carry

    lax.fori_loop(0, 2, row_body, 0)


def _radix_sort(u_flat):
    mesh = plsc.VectorSubcoreMesh(core_axis_name="c", subcore_axis_name="s")
    run = pl.kernel(
        _sort_body,
        mesh=mesh,
        compiler_params=pltpu.CompilerParams(needs_layout_passes=False),
        out_type=jax.ShapeDtypeStruct((B * HW,), jnp.int32),
        scratch_types=[
            pltpu.VMEM((CHK,), jnp.int32),
            pltpu.VMEM((CHK,), jnp.int32),
            pltpu.VMEM((CHK,), jnp.int32),
            pltpu.VMEM((CHK,), jnp.int32),
            pltpu.VMEM((CHK,), jnp.int32),
            pltpu.VMEM((RADIX * 16,), jnp.int32),
            pltpu.VMEM((RADIX * 16,), jnp.int32),
            pltpu.VMEM((RADIX * 16 * NTILES,), jnp.int32),
            pltpu.SMEM((RADIX,), jnp.int32),
            pltpu.SMEM((RADIX,), jnp.int32),
            pltpu.VMEM_SHARED((HW,), jnp.int32),
            pltpu.VMEM_SHARED((HW,), jnp.int32),
            pltpu.VMEM_SHARED((HW,), jnp.int32),
            pltpu.VMEM_SHARED((HW,), jnp.int32),
            pltpu.VMEM_SHARED((RADIX * 16 * NTILES,), jnp.int32),
        ],
    )
    return run(u_flat)


def _gather_body(f_hbm, t_hbm, idx_hbm, out_f, out_t,
                 idx_v, vals_v, tvals_v, row_sh, trow_sh):
    cid = lax.axis_index("c")
    sid = lax.axis_index("s")

    for j in range(2):
        b = cid * 2 + j
        pltpu.sync_copy(idx_hbm.at[pl.ds(_al(b * NP + sid * CH), CH)], idx_v)

        @pl.when(sid == 0)
        def _():
            pltpu.sync_copy(t_hbm.at[pl.ds(_al(b * HW), HW)], trow_sh)
        plsc.subcore_barrier()
        pltpu.sync_copy(trow_sh.at[idx_v], tvals_v)
        pltpu.sync_copy(tvals_v, out_t.at[pl.ds(_al(b * NP + sid * CH), CH)])

        def chan(c, carry):
            @pl.when(sid == 0)
            def _():
                pltpu.sync_copy(f_hbm.at[pl.ds(_al((b * C + c) * HW), HW)],
                                row_sh)
            plsc.subcore_barrier()
            pltpu.sync_copy(row_sh.at[idx_v], vals_v)
            pltpu.sync_copy(
                vals_v,
                out_f.at[pl.ds(_al((b * C + c) * NP + sid * CH), CH)])
            plsc.subcore_barrier()
            return carry

        lax.fori_loop(0, C, chan, 0)


def _sample_gather(f_flat1, t_flat1, idx1):
    mesh = plsc.VectorSubcoreMesh(core_axis_name="c", subcore_axis_name="s")
    run = pl.kernel(
        _gather_body,
        mesh=mesh,
        out_type=(
            jax.ShapeDtypeStruct((B * C * NP,), jnp.float32),
            jax.ShapeDtypeStruct((B * NP,), jnp.int32),
        ),
        scratch_types=[
            pltpu.VMEM((CH,), jnp.int32),
            pltpu.VMEM((CH,), jnp.float32),
            pltpu.VMEM((CH,), jnp.int32),
            pltpu.VMEM_SHARED((HW,), jnp.float32),
            pltpu.VMEM_SHARED((HW,), jnp.int32),
        ],
    )
    return run(f_flat1, t_flat1, idx1)


from jax.experimental.compute_on import compute_on


@compute_on("tpu_sparsecore")
@jax.jit
def _sc_probs(cw_big, idx2):
    return cw_big[idx2].astype(jnp.float32)


def kernel(features, targets, sampling_weights):
    total = float(targets.size)
    flat_t = targets.reshape(-1)
    counts = jnp.bincount(flat_t, length=NCLS)
    num_classes = jnp.sum(counts > 0).astype(jnp.float32)
    counts_f = jnp.maximum(counts.astype(jnp.float32), 1.0)
    class_w = jnp.where(counts > 0, total / (num_classes * counts_f), 1.0)
    krep = 512
    cw_big = jnp.tile(class_w, krep)
    off = (jnp.arange(targets.size, dtype=jnp.int32) % krep).reshape(targets.shape)
    probs = _sc_probs(cw_big, targets + NCLS * off)
    probs = probs / probs.sum()
    probs = probs * sampling_weights[0]
    p_flat = probs.reshape(B, HW)

    u = jax.random.uniform(jax.random.key(42), (B, HW), minval=1e-20, maxval=1.0)
    g = -jnp.log(-jnp.log(u))
    keys_g = jnp.log(jnp.maximum(p_flat, 1e-30)) + g

    kb = lax.bitcast_convert_type(keys_g, jnp.int32)
    u_desc = jnp.where(kb >= 0, ~(kb ^ jnp.int32(-2**31)), kb)
    sorted_idx = _radix_sort(u_desc.reshape(-1))
    idx_in = sorted_idx.reshape(B, HW)[:, :NP].reshape(-1)

    f_flat1 = features.reshape(-1)
    t_flat1 = targets.reshape(-1)
    out_f, out_t = _sample_gather(f_flat1, t_flat1, idx_in)
    sf = out_f.reshape(B, C, NP)[:, :, :NKEEP].reshape(B, C, NH, NH)
    st = out_t.reshape(B, NP)[:, :NKEEP].reshape(B, NH, NH)
    return sf, st
